# baseline (speedup 1.0000x reference)
import functools

import jax
import jax.numpy as jnp
from jax import lax
from jax.experimental import pallas as pl
from jax.experimental.pallas import tpu as pltpu
from jax.experimental.pallas import tpu_sc as plsc

S, D = 2048, 1024
H, HD = 16, 64
E, K, HID = 8, 2, 2048

TB = 256
NBT = (K * S) // TB + E
MPAD = NBT * TB

NC, NS = 2, 16
NW = NC * NS
A_PER_W = (K * S) // NW
T_PER_W = S // NW

HIGHEST = jax.lax.Precision.HIGHEST
BF = jnp.bfloat16
F32 = jnp.float32


def _split(x):
    hi = x.astype(BF)
    lo = (x - hi.astype(F32)).astype(BF)
    return hi, lo


def _mm3(a_hi, a_lo, b_hi, b_lo, dims):
    dn = (dims, ((), ()))
    t = jax.lax.dot_general(a_hi, b_lo, dn, preferred_element_type=F32)
    t = t + jax.lax.dot_general(a_lo, b_hi, dn, preferred_element_type=F32)
    t = t + jax.lax.dot_general(a_hi, b_hi, dn, preferred_element_type=F32)
    return t


def _ln_f32(x, w, b):
    m = jnp.mean(x, axis=-1, keepdims=True)
    v = jnp.mean((x - m) ** 2, axis=-1, keepdims=True)
    return (x - m) / jnp.sqrt(v + 1e-5) * w + b


def _ln_qkv_body(x_ref, whi_ref, wlo_ref, lnw_ref, lnb_ref, b_ref,
                 hi_ref, lo_ref):
    h = _ln_f32(x_ref[...], lnw_ref[...], lnb_ref[...])
    h_hi, h_lo = _split(h)
    out = _mm3(h_hi, h_lo, whi_ref[...], wlo_ref[...], ((1,), (0,)))
    out = out + b_ref[...]
    o_hi, o_lo = _split(out)
    hi_ref[...] = o_hi
    lo_ref[...] = o_lo


def _ln_qkv(x, qkv_W, qkv_b, ln1_w, ln1_b):
    SB, NB = 256, 1024
    w_hi, w_lo = _split(qkv_W)
    return pl.pallas_call(
        _ln_qkv_body,
        grid=(3, S // SB),
        in_specs=[
            pl.BlockSpec((SB, D), lambda nb, sb: (sb, 0)),
            pl.BlockSpec((D, NB), lambda nb, sb: (0, nb)),
            pl.BlockSpec((D, NB), lambda nb, sb: (0, nb)),
            pl.BlockSpec((D,), lambda nb, sb: (0,)),
            pl.BlockSpec((D,), lambda nb, sb: (0,)),
            pl.BlockSpec((NB,), lambda nb, sb: (nb,)),
        ],
        out_specs=[
            pl.BlockSpec((SB, NB), lambda nb, sb: (sb, nb)),
            pl.BlockSpec((SB, NB), lambda nb, sb: (sb, nb)),
        ],
        out_shape=[
            jax.ShapeDtypeStruct((S, 3 * D), BF),
            jax.ShapeDtypeStruct((S, 3 * D), BF),
        ],
    )(x, w_hi, w_lo, ln1_w, ln1_b, qkv_b)


def _attn_body(qhi_ref, qlo_ref, khi_ref, klo_ref, vhi_ref, vlo_ref,
               out_ref):
    pieces = []
    for h in range(H):
        sl = slice(h * HD, (h + 1) * HD)
        q_hi = qhi_ref[:, sl]
        q_lo = qlo_ref[:, sl]
        k_hi = khi_ref[:, sl]
        k_lo = klo_ref[:, sl]
        v_hi = vhi_ref[:, sl]
        v_lo = vlo_ref[:, sl]
        sT = _mm3(k_hi, k_lo, q_hi, q_lo, ((1,), (1,))) * (HD ** -0.5)
        m = jnp.max(sT, axis=0, keepdims=True)
        p = jnp.exp(sT - m)
        rec = 1.0 / jnp.sum(p, axis=0, keepdims=True)
        p_hi, p_lo = _split(p)
        aoT = _mm3(v_hi, v_lo, p_hi, p_lo, ((0,), (0,)))
        pieces.append(aoT * rec)
    out_ref[...] = jnp.concatenate(pieces, axis=0)


def _attention(qkv_hi, qkv_lo):
    QB = 512
    return pl.pallas_call(
        _attn_body,
        grid=(S // QB,),
        in_specs=[
            pl.BlockSpec((QB, D), lambda qb: (qb, 0)),
            pl.BlockSpec((QB, D), lambda qb: (qb, 0)),
            pl.BlockSpec((S, D), lambda qb: (0, 1)),
            pl.BlockSpec((S, D), lambda qb: (0, 1)),
            pl.BlockSpec((S, D), lambda qb: (0, 2)),
            pl.BlockSpec((S, D), lambda qb: (0, 2)),
        ],
        out_specs=pl.BlockSpec((D, QB), lambda qb: (0, qb)),
        out_shape=jax.ShapeDtypeStruct((D, S), F32),
    )(qkv_hi, qkv_lo, qkv_hi, qkv_lo, qkv_hi, qkv_lo)


def _proj_router_body(aoT_ref, x_ref, whi_ref, wlo_ref, b_ref,
                      ln2w_ref, ln2b_ref, gw_ref, gb_ref,
                      x2_ref, h2_ref, wk_ref, idx2_ref):
    ao = aoT_ref[...].T
    a_hi, a_lo = _split(ao)
    proj = _mm3(a_hi, a_lo, whi_ref[...], wlo_ref[...], ((1,), (0,)))
    x2 = x_ref[...] + proj + b_ref[...]
    x2_ref[...] = x2
    h2 = _ln_f32(x2, ln2w_ref[...], ln2b_ref[...])
    h2_ref[...] = h2
    g = jnp.dot(h2, gw_ref[...], precision=HIGHEST,
                preferred_element_type=F32) + gb_ref[...]
    iota = jax.lax.broadcasted_iota(jnp.int32, g.shape, 1)
    m1 = jnp.max(g, axis=-1, keepdims=True)
    i1 = jnp.min(jnp.where(g == m1, iota, E), axis=-1, keepdims=True)
    g2 = jnp.where(iota == i1, -jnp.inf, g)
    m2 = jnp.max(g2, axis=-1, keepdims=True)
    i2 = jnp.min(jnp.where(g2 == m2, iota, E), axis=-1, keepdims=True)
    p1 = 1.0 / (1.0 + jnp.exp(m2 - m1))
    p2 = 1.0 / (1.0 + jnp.exp(m1 - m2))
    wk_ref[...] = jnp.concatenate([p1, p2], axis=1)
    idx2_ref[...] = jnp.concatenate([i1, i2], axis=1)


def _proj_router(aoT, x, attn_out_W, attn_out_b, ln2_w, ln2_b,
                 gate_W, gate_b):
    SB = 256
    w_hi, w_lo = _split(attn_out_W)
    return pl.pallas_call(
        _proj_router_body,
        grid=(S // SB,),
        in_specs=[
            pl.BlockSpec((D, SB), lambda sb: (0, sb)),
            pl.BlockSpec((SB, D), lambda sb: (sb, 0)),
            pl.BlockSpec((D, D), lambda sb: (0, 0)),
            pl.BlockSpec((D, D), lambda sb: (0, 0)),
            pl.BlockSpec((D,), lambda sb: (0,)),
            pl.BlockSpec((D,), lambda sb: (0,)),
            pl.BlockSpec((D,), lambda sb: (0,)),
            pl.BlockSpec((D, E), lambda sb: (0, 0)),
            pl.BlockSpec((E,), lambda sb: (0,)),
        ],
        out_specs=[
            pl.BlockSpec((SB, D), lambda sb: (sb, 0)),
            pl.BlockSpec((SB, D), lambda sb: (sb, 0)),
            pl.BlockSpec((SB, K), lambda sb: (sb, 0)),
            pl.BlockSpec((SB, K), lambda sb: (sb, 0)),
        ],
        out_shape=[
            jax.ShapeDtypeStruct((S, D), F32),
            jax.ShapeDtypeStruct((S, D), F32),
            jax.ShapeDtypeStruct((S, K), F32),
            jax.ShapeDtypeStruct((S, K), jnp.int32),
        ],
    )(aoT, x, w_hi, w_lo, attn_out_b, ln2_w, ln2_b, gate_W, gate_b)


def _route_meta_body(idx2_ref, dest_ref, bexp_ref, blive_ref):
    idx = idx2_ref[...]
    i1 = idx[:, 0:1]
    i2 = idx[:, 1:2]
    eio = jax.lax.broadcasted_iota(jnp.int32, (S, E), 1)
    oh1 = (eio == i1)
    oh2 = (eio == i2)
    sel = oh1.astype(BF) + oh2.astype(BF)
    MB = 512
    cum = jnp.zeros((S, E), F32)
    for b in range(S // MB):
        row = jax.lax.broadcasted_iota(jnp.int32, (S, MB), 0)
        col = jax.lax.broadcasted_iota(jnp.int32, (S, MB), 1) + b * MB
        ltb = (col < row).astype(BF)
        cum = cum + jnp.dot(ltb, sel[b * MB:(b + 1) * MB, :],
                            preferred_element_type=F32)
    count = cum[S - 1:S, :] + sel[S - 1:S, :].astype(F32)
    nblk = jnp.floor((count + (TB - 1)) * (1.0 / TB))
    upper = (jax.lax.broadcasted_iota(jnp.int32, (E, E), 0)
             < jax.lax.broadcasted_iota(jnp.int32, (E, E), 1))
    blk_excl = jnp.dot(nblk.astype(BF), upper.astype(BF),
                       preferred_element_type=F32)
    gstart = blk_excl * float(TB)
    base = gstart + cum
    d1 = jnp.sum(jnp.where(oh1, base, 0.0), axis=1, keepdims=True)
    d2 = jnp.sum(jnp.where(oh2, base, 0.0), axis=1, keepdims=True)
    dest_ref[...] = jnp.concatenate([d1.T, d2.T], axis=0).astype(jnp.int32)
    bx = blk_excl.T.astype(jnp.int32)
    nb = nblk.T.astype(jnp.int32)
    jot = jax.lax.broadcasted_iota(jnp.int32, (E, NBT), 1)
    mask = jnp.logical_and(bx <= jot, jot < bx + nb)
    ecol = jax.lax.broadcasted_iota(jnp.int32, (E, NBT), 0)
    bexp_live = jnp.sum(jnp.where(mask, ecol, 0), axis=0, keepdims=True)
    blive = jnp.sum(mask.astype(jnp.int32), axis=0, keepdims=True)
    eiota = jax.lax.broadcasted_iota(jnp.int32, (1, E), 1)
    last_e = jnp.max(jnp.where(count > 0, eiota, 0))
    bexp_ref[...] = jnp.where(blive > 0, bexp_live, last_e)
    blive_ref[...] = blive


def _route_meta(idx2):
    return pl.pallas_call(
        _route_meta_body,
        grid=(1,),
        in_specs=[pl.BlockSpec((S, K), lambda i: (0, 0))],
        out_specs=[
            pl.BlockSpec((K, S), lambda i: (0, 0)),
            pl.BlockSpec((1, NBT), lambda i: (0, 0)),
            pl.BlockSpec((1, NBT), lambda i: (0, 0)),
        ],
        out_shape=[
            jax.ShapeDtypeStruct((K, S), jnp.int32),
            jax.ShapeDtypeStruct((1, NBT), jnp.int32),
            jax.ShapeDtypeStruct((1, NBT), jnp.int32),
        ],
    )(idx2)


def _sc_mesh():
    return plsc.VectorSubcoreMesh(core_axis_name="c", subcore_axis_name="s")


NCH = 4
CH = A_PER_W // NCH


def _dispatch(h2f, dest_flat):
    @functools.partial(
        pl.kernel,
        out_type=jax.ShapeDtypeStruct((MPAD, D), F32),
        mesh=_sc_mesh(),
        scratch_types=[
            pltpu.VMEM((NCH, CH), jnp.int32),
            pltpu.VMEM((NCH, CH), jnp.int32),
            pltpu.VMEM((CH, D), F32),
            pltpu.VMEM((CH, D), F32),
            pltpu.SemaphoreType.DMA,
            pltpu.SemaphoreType.DMA,
            pltpu.SemaphoreType.DMA,
            pltpu.SemaphoreType.DMA,
        ],
    )
    def k(h2_hbm, dest_hbm, xs_hbm, didx, sidx, rows0, rows1,
          g0, g1, s0, s1):
        wid = lax.axis_index("s") * NC + lax.axis_index("c")
        base = wid * A_PER_W
        bufs = (rows0, rows1)
        gsems = (g0, g1)
        ssems = (s0, s1)
        for c in range(NCH):
            pltpu.sync_copy(dest_hbm.at[pl.ds(base + c * CH, CH)],
                            didx.at[c])
            for j in range(CH // 16):
                it = lax.iota(jnp.int32, 16) + (c * CH + j * 16)
                sidx[c, pl.ds(j * 16, 16)] = jnp.bitwise_and(
                    base + it, S - 1)
        scat = [None, None]
        for c in range(NCH):
            b = c & 1
            if scat[b] is not None:
                scat[b].wait()
            pltpu.async_copy(h2_hbm.at[sidx.at[c]], bufs[b],
                             gsems[b]).wait()
            scat[b] = pltpu.async_copy(bufs[b], xs_hbm.at[didx.at[c]],
                                       ssems[b])
        scat[0].wait()
        scat[1].wait()

    return k(h2f, dest_flat)


def _gelu_exact(u):
    return u * 0.5 * (1.0 + jax.lax.erf(u * (2.0 ** -0.5)))


def _moe_ffn_body(bexp_ref, blive_ref, xs_ref, w1_ref, b1_ref, w2_ref,
                  b2_ref, y_ref):
    j = pl.program_id(0)

    @pl.when(blive_ref[j] > 0)
    def _compute():
        u = jnp.dot(xs_ref[...].astype(BF), w1_ref[0],
                    preferred_element_type=F32)
        u = u + b1_ref[0]
        hid = _gelu_exact(u).astype(BF)
        y_ref[...] = (jnp.dot(hid, w2_ref[0], preferred_element_type=F32)
                      + b2_ref[0])


def _moe_ffn(xs, bexp, blive, exp_W1b, exp_b1, exp_W2b, exp_b2):
    grid_spec = pltpu.PrefetchScalarGridSpec(
        num_scalar_prefetch=2,
        grid=(NBT,),
        in_specs=[
            pl.BlockSpec((TB, D), lambda j, be, bl: (j, 0)),
            pl.BlockSpec((1, D, HID), lambda j, be, bl: (be[j], 0, 0)),
            pl.BlockSpec((1, 1, HID), lambda j, be, bl: (be[j], 0, 0)),
            pl.BlockSpec((1, HID, D), lambda j, be, bl: (be[j], 0, 0)),
            pl.BlockSpec((1, 1, D), lambda j, be, bl: (be[j], 0, 0)),
        ],
        out_specs=pl.BlockSpec((TB, D), lambda j, be, bl: (j, 0)),
    )
    return pl.pallas_call(
        _moe_ffn_body,
        grid_spec=grid_spec,
        out_shape=jax.ShapeDtypeStruct((MPAD, D), F32),
        compiler_params=pltpu.CompilerParams(
            dimension_semantics=("arbitrary",)),
    )(bexp, blive, xs, exp_W1b, exp_b1, exp_W2b, exp_b2)


def _combine_gather(y, dest_flat):
    @functools.partial(
        pl.kernel,
        out_type=[
            jax.ShapeDtypeStruct((S, D), F32),
            jax.ShapeDtypeStruct((S, D), F32),
        ],
        mesh=_sc_mesh(),
        scratch_types=[
            pltpu.VMEM((T_PER_W,), jnp.int32),
            pltpu.VMEM((T_PER_W, D), F32),
            pltpu.SemaphoreType.DMA,
        ],
    )
    def k(y_hbm, dest_hbm, y0_hbm, y1_hbm, didx_v, rows_v, sem):
        wid = lax.axis_index("s") * NC + lax.axis_index("c")
        base_t = wid * T_PER_W
        for kk, out_hbm in ((0, y0_hbm), (1, y1_hbm)):
            pltpu.sync_copy(
                dest_hbm.at[pl.ds(kk * S + base_t, T_PER_W)], didx_v)
            pltpu.async_copy(y_hbm.at[didx_v], rows_v, sem).wait()
            pltpu.sync_copy(rows_v, out_hbm.at[pl.ds(base_t, T_PER_W)])

    return k(y, dest_flat)


def _final_body(x2_ref, y0_ref, y1_ref, wk_ref, w_ref, b_ref, out_ref):
    w0 = wk_ref[...][:, 0:1]
    w1 = wk_ref[...][:, 1:2]
    z = x2_ref[...] + w0 * y0_ref[...] + w1 * y1_ref[...]
    m = jnp.sum(z, axis=0, keepdims=True) * (1.0 / S)
    out_ref[...] = jnp.dot(m, w_ref[...], precision=HIGHEST,
                           preferred_element_type=F32) + b_ref[...]


def _final(x2, y0, y1, wk, fc_W, fc_b):
    return pl.pallas_call(
        _final_body,
        grid=(1,),
        in_specs=[
            pl.BlockSpec((S, D), lambda i: (0, 0)),
            pl.BlockSpec((S, D), lambda i: (0, 0)),
            pl.BlockSpec((S, D), lambda i: (0, 0)),
            pl.BlockSpec((S, K), lambda i: (0, 0)),
            pl.BlockSpec((D, D), lambda i: (0, 0)),
            pl.BlockSpec((D,), lambda i: (0,)),
        ],
        out_specs=pl.BlockSpec((1, D), lambda i: (0, 0)),
        out_shape=jax.ShapeDtypeStruct((1, D), F32),
    )(x2, y0, y1, wk, fc_W, fc_b)


def kernel(x, qkv_W, qkv_b, attn_out_W, attn_out_b, gate_W, gate_b,
           exp_W1, exp_b1, exp_W2, exp_b2, ln1_w, ln1_b, ln2_w, ln2_b,
           fc_W, fc_b):
    xs = x.reshape(S, D)
    qkv_hi, qkv_lo = _ln_qkv(xs, qkv_W, qkv_b, ln1_w, ln1_b)
    aoT = _attention(qkv_hi, qkv_lo)
    x2, h2f, wk, idx2 = _proj_router(aoT, xs, attn_out_W, attn_out_b,
                                     ln2_w, ln2_b, gate_W, gate_b)
    dest, bexp, blive = _route_meta(idx2)
    dest_flat = dest.reshape(K * S)
    xs_sorted = _dispatch(h2f, dest_flat)
    y = _moe_ffn(xs_sorted, bexp.reshape(NBT), blive.reshape(NBT),
                 exp_W1.astype(BF), exp_b1.reshape(E, 1, HID),
                 exp_W2.astype(BF), exp_b2.reshape(E, 1, D))
    y0, y1 = _combine_gather(y, dest_flat)
    return _final(x2, y0, y1, wk, fc_W, fc_b)


# speedup vs baseline: 1.0959x; 1.0959x over previous
import functools

import jax
import jax.numpy as jnp
from jax import lax
from jax.experimental import pallas as pl
from jax.experimental.pallas import tpu as pltpu
from jax.experimental.pallas import tpu_sc as plsc

S, D = 2048, 1024
H, HD = 16, 64
E, K, HID = 8, 2, 2048

TB = 256
NBT = (K * S) // TB + E
MPAD = NBT * TB

NC, NS = 2, 16
NW = NC * NS
A_PER_W = (K * S) // NW
T_PER_W = S // NW

HIGHEST = jax.lax.Precision.HIGHEST
BF = jnp.bfloat16
F32 = jnp.float32


def _split(x):
    hi = x.astype(BF)
    lo = (x - hi.astype(F32)).astype(BF)
    return hi, lo


def _mm3(a_hi, a_lo, b_hi, b_lo, dims):
    dn = (dims, ((), ()))
    t = jax.lax.dot_general(a_hi, b_lo, dn, preferred_element_type=F32)
    t = t + jax.lax.dot_general(a_lo, b_hi, dn, preferred_element_type=F32)
    t = t + jax.lax.dot_general(a_hi, b_hi, dn, preferred_element_type=F32)
    return t


def _ln_f32(x, w, b):
    m = jnp.mean(x, axis=-1, keepdims=True)
    v = jnp.mean((x - m) ** 2, axis=-1, keepdims=True)
    return (x - m) / jnp.sqrt(v + 1e-5) * w + b


def _ln_qkv_body(x_ref, w_ref, lnw_ref, lnb_ref, b_ref, hi_ref, lo_ref):
    h = _ln_f32(x_ref[...], lnw_ref[...], lnb_ref[...])
    h_hi, h_lo = _split(h)
    w_hi, w_lo = _split(w_ref[...])
    out = _mm3(h_hi, h_lo, w_hi, w_lo, ((1,), (0,)))
    out = out + b_ref[...]
    o_hi, o_lo = _split(out)
    hi_ref[...] = o_hi
    lo_ref[...] = o_lo


def _ln_qkv(x, qkv_W, qkv_b, ln1_w, ln1_b):
    SB, NB = 256, 1024
    return pl.pallas_call(
        _ln_qkv_body,
        grid=(3, S // SB),
        in_specs=[
            pl.BlockSpec((SB, D), lambda nb, sb: (sb, 0)),
            pl.BlockSpec((D, NB), lambda nb, sb: (0, nb)),
            pl.BlockSpec((D,), lambda nb, sb: (0,)),
            pl.BlockSpec((D,), lambda nb, sb: (0,)),
            pl.BlockSpec((NB,), lambda nb, sb: (nb,)),
        ],
        out_specs=[
            pl.BlockSpec((SB, NB), lambda nb, sb: (sb, nb)),
            pl.BlockSpec((SB, NB), lambda nb, sb: (sb, nb)),
        ],
        out_shape=[
            jax.ShapeDtypeStruct((S, 3 * D), BF),
            jax.ShapeDtypeStruct((S, 3 * D), BF),
        ],
    )(x, qkv_W, ln1_w, ln1_b, qkv_b)


def _attn_body(qhi_ref, qlo_ref, khi_ref, klo_ref, vhi_ref, vlo_ref,
               out_ref):
    pieces = []
    for h in range(H):
        sl = slice(h * HD, (h + 1) * HD)
        q_hi = qhi_ref[:, sl]
        q_lo = qlo_ref[:, sl]
        k_hi = khi_ref[:, sl]
        k_lo = klo_ref[:, sl]
        v_hi = vhi_ref[:, sl]
        v_lo = vlo_ref[:, sl]
        sT = _mm3(k_hi, k_lo, q_hi, q_lo, ((1,), (1,))) * (HD ** -0.5)
        m = jnp.max(sT, axis=0, keepdims=True)
        p = jnp.exp(sT - m)
        rec = 1.0 / jnp.sum(p, axis=0, keepdims=True)
        p_hi, p_lo = _split(p)
        aoT = _mm3(v_hi, v_lo, p_hi, p_lo, ((0,), (0,)))
        pieces.append(aoT * rec)
    out_ref[...] = jnp.concatenate(pieces, axis=0)


def _attention(qkv_hi, qkv_lo):
    QB = 512
    return pl.pallas_call(
        _attn_body,
        grid=(S // QB,),
        in_specs=[
            pl.BlockSpec((QB, D), lambda qb: (qb, 0)),
            pl.BlockSpec((QB, D), lambda qb: (qb, 0)),
            pl.BlockSpec((S, D), lambda qb: (0, 1)),
            pl.BlockSpec((S, D), lambda qb: (0, 1)),
            pl.BlockSpec((S, D), lambda qb: (0, 2)),
            pl.BlockSpec((S, D), lambda qb: (0, 2)),
        ],
        out_specs=pl.BlockSpec((D, QB), lambda qb: (0, qb)),
        out_shape=jax.ShapeDtypeStruct((D, S), F32),
    )(qkv_hi, qkv_lo, qkv_hi, qkv_lo, qkv_hi, qkv_lo)


def _proj_router_body(aoT_ref, x_ref, w_ref, b_ref,
                      ln2w_ref, ln2b_ref, gw_ref, gb_ref,
                      x2_ref, h2_ref, wk_ref, dest_ref, bexp_ref,
                      blive_ref):
    ao = aoT_ref[...].T
    a_hi, a_lo = _split(ao)
    w_hi, w_lo = _split(w_ref[...])
    proj = _mm3(a_hi, a_lo, w_hi, w_lo, ((1,), (0,)))
    x2 = x_ref[...] + proj + b_ref[...]
    x2_ref[...] = x2
    h2 = _ln_f32(x2, ln2w_ref[...], ln2b_ref[...])
    h2_ref[...] = h2
    g = jnp.dot(h2, gw_ref[...], precision=HIGHEST,
                preferred_element_type=F32) + gb_ref[...]
    iota = jax.lax.broadcasted_iota(jnp.int32, g.shape, 1)
    m1 = jnp.max(g, axis=-1, keepdims=True)
    i1 = jnp.min(jnp.where(g == m1, iota, E), axis=-1, keepdims=True)
    g2 = jnp.where(iota == i1, -jnp.inf, g)
    m2 = jnp.max(g2, axis=-1, keepdims=True)
    i2 = jnp.min(jnp.where(g2 == m2, iota, E), axis=-1, keepdims=True)
    p1 = 1.0 / (1.0 + jnp.exp(m2 - m1))
    p2 = 1.0 / (1.0 + jnp.exp(m1 - m2))
    wk_ref[...] = jnp.concatenate([p1, p2], axis=1)
    eio = jax.lax.broadcasted_iota(jnp.int32, (S, E), 1)
    oh1 = (eio == i1)
    oh2 = (eio == i2)
    sel = oh1.astype(BF) + oh2.astype(BF)
    MB = 512
    cum = jnp.zeros((S, E), F32)
    for b in range(S // MB):
        row = jax.lax.broadcasted_iota(jnp.int32, (S, MB), 0)
        col = jax.lax.broadcasted_iota(jnp.int32, (S, MB), 1) + b * MB
        ltb = (col < row).astype(BF)
        cum = cum + jnp.dot(ltb, sel[b * MB:(b + 1) * MB, :],
                            preferred_element_type=F32)
    count = cum[S - 1:S, :] + sel[S - 1:S, :].astype(F32)
    nblk = jnp.floor((count + (TB - 1)) * (1.0 / TB))
    upper = (jax.lax.broadcasted_iota(jnp.int32, (E, E), 0)
             < jax.lax.broadcasted_iota(jnp.int32, (E, E), 1))
    blk_excl = jnp.dot(nblk.astype(BF), upper.astype(BF),
                       preferred_element_type=F32)
    gstart = blk_excl * float(TB)
    base = gstart + cum
    d1 = jnp.sum(jnp.where(oh1, base, 0.0), axis=1, keepdims=True)
    d2 = jnp.sum(jnp.where(oh2, base, 0.0), axis=1, keepdims=True)
    dest_ref[...] = jnp.concatenate([d1.T, d2.T], axis=0).astype(jnp.int32)
    bx = blk_excl.T.astype(jnp.int32)
    nb = nblk.T.astype(jnp.int32)
    jot = jax.lax.broadcasted_iota(jnp.int32, (E, NBT), 1)
    mask = jnp.logical_and(bx <= jot, jot < bx + nb)
    ecol = jax.lax.broadcasted_iota(jnp.int32, (E, NBT), 0)
    bexp_live = jnp.sum(jnp.where(mask, ecol, 0), axis=0, keepdims=True)
    blive = jnp.sum(mask.astype(jnp.int32), axis=0, keepdims=True)
    eiota = jax.lax.broadcasted_iota(jnp.int32, (1, E), 1)
    last_e = jnp.max(jnp.where(count > 0, eiota, 0))
    bexp_ref[...] = jnp.where(blive > 0, bexp_live, last_e)
    blive_ref[...] = blive


def _proj_router(aoT, x, attn_out_W, attn_out_b, ln2_w, ln2_b,
                 gate_W, gate_b):
    return pl.pallas_call(
        _proj_router_body,
        grid=(1,),
        in_specs=[
            pl.BlockSpec((D, S), lambda i: (0, 0)),
            pl.BlockSpec((S, D), lambda i: (0, 0)),
            pl.BlockSpec((D, D), lambda i: (0, 0)),
            pl.BlockSpec((D,), lambda i: (0,)),
            pl.BlockSpec((D,), lambda i: (0,)),
            pl.BlockSpec((D,), lambda i: (0,)),
            pl.BlockSpec((D, E), lambda i: (0, 0)),
            pl.BlockSpec((E,), lambda i: (0,)),
        ],
        out_specs=[
            pl.BlockSpec((S, D), lambda i: (0, 0)),
            pl.BlockSpec((S, D), lambda i: (0, 0)),
            pl.BlockSpec((S, K), lambda i: (0, 0)),
            pl.BlockSpec((K, S), lambda i: (0, 0)),
            pl.BlockSpec((1, NBT), lambda i: (0, 0)),
            pl.BlockSpec((1, NBT), lambda i: (0, 0)),
        ],
        out_shape=[
            jax.ShapeDtypeStruct((S, D), F32),
            jax.ShapeDtypeStruct((S, D), F32),
            jax.ShapeDtypeStruct((S, K), F32),
            jax.ShapeDtypeStruct((K, S), jnp.int32),
            jax.ShapeDtypeStruct((1, NBT), jnp.int32),
            jax.ShapeDtypeStruct((1, NBT), jnp.int32),
        ],
    )(aoT, x, attn_out_W, attn_out_b, ln2_w, ln2_b, gate_W, gate_b)


def _sc_mesh():
    return plsc.VectorSubcoreMesh(core_axis_name="c", subcore_axis_name="s")


NCH = 4
CH = A_PER_W // NCH


def _dispatch(h2f, dest_flat):
    @functools.partial(
        pl.kernel,
        out_type=jax.ShapeDtypeStruct((MPAD, D), F32),
        mesh=_sc_mesh(),
        scratch_types=[
            pltpu.VMEM((NCH, CH), jnp.int32),
            pltpu.VMEM((NCH, CH), jnp.int32),
            pltpu.VMEM((CH, D), F32),
            pltpu.VMEM((CH, D), F32),
            pltpu.SemaphoreType.DMA,
            pltpu.SemaphoreType.DMA,
            pltpu.SemaphoreType.DMA,
            pltpu.SemaphoreType.DMA,
        ],
    )
    def k(h2_hbm, dest_hbm, xs_hbm, didx, sidx, rows0, rows1,
          g0, g1, s0, s1):
        wid = lax.axis_index("s") * NC + lax.axis_index("c")
        base = wid * A_PER_W
        bufs = (rows0, rows1)
        gsems = (g0, g1)
        ssems = (s0, s1)
        for c in range(NCH):
            pltpu.sync_copy(dest_hbm.at[pl.ds(base + c * CH, CH)],
                            didx.at[c])
            for j in range(CH // 16):
                it = lax.iota(jnp.int32, 16) + (c * CH + j * 16)
                sidx[c, pl.ds(j * 16, 16)] = jnp.bitwise_and(
                    base + it, S - 1)
        scat = [None, None]
        for c in range(NCH):
            b = c & 1
            if scat[b] is not None:
                scat[b].wait()
            pltpu.async_copy(h2_hbm.at[sidx.at[c]], bufs[b],
                             gsems[b]).wait()
            scat[b] = pltpu.async_copy(bufs[b], xs_hbm.at[didx.at[c]],
                                       ssems[b])
        scat[0].wait()
        scat[1].wait()

    return k(h2f, dest_flat)


def _gelu_exact(u):
    return u * 0.5 * (1.0 + jax.lax.erf(u * (2.0 ** -0.5)))


def _moe_ffn_body(bexp_ref, blive_ref, xs_ref, w1_ref, b1_ref, w2_ref,
                  b2_ref, y_ref):
    j = pl.program_id(0)

    @pl.when(blive_ref[j] > 0)
    def _compute():
        u = jnp.dot(xs_ref[...].astype(BF), w1_ref[0].astype(BF),
                    preferred_element_type=F32)
        u = u + b1_ref[0]
        hid = _gelu_exact(u).astype(BF)
        y_ref[...] = (jnp.dot(hid, w2_ref[0].astype(BF),
                              preferred_element_type=F32) + b2_ref[0])


def _moe_ffn(xs, bexp, blive, exp_W1b, exp_b1, exp_W2b, exp_b2):
    grid_spec = pltpu.PrefetchScalarGridSpec(
        num_scalar_prefetch=2,
        grid=(NBT,),
        in_specs=[
            pl.BlockSpec((TB, D), lambda j, be, bl: (j, 0)),
            pl.BlockSpec((1, D, HID), lambda j, be, bl: (be[j], 0, 0)),
            pl.BlockSpec((1, 1, HID), lambda j, be, bl: (be[j], 0, 0)),
            pl.BlockSpec((1, HID, D), lambda j, be, bl: (be[j], 0, 0)),
            pl.BlockSpec((1, 1, D), lambda j, be, bl: (be[j], 0, 0)),
        ],
        out_specs=pl.BlockSpec((TB, D), lambda j, be, bl: (j, 0)),
    )
    return pl.pallas_call(
        _moe_ffn_body,
        grid_spec=grid_spec,
        out_shape=jax.ShapeDtypeStruct((MPAD, D), F32),
        compiler_params=pltpu.CompilerParams(
            dimension_semantics=("arbitrary",)),
    )(bexp, blive, xs, exp_W1b, exp_b1, exp_W2b, exp_b2)


def _combine_gather(y, dest_flat):
    @functools.partial(
        pl.kernel,
        out_type=[
            jax.ShapeDtypeStruct((S, D), F32),
            jax.ShapeDtypeStruct((S, D), F32),
        ],
        mesh=_sc_mesh(),
        scratch_types=[
            pltpu.VMEM((T_PER_W,), jnp.int32),
            pltpu.VMEM((T_PER_W, D), F32),
            pltpu.SemaphoreType.DMA,
        ],
    )
    def k(y_hbm, dest_hbm, y0_hbm, y1_hbm, didx_v, rows_v, sem):
        wid = lax.axis_index("s") * NC + lax.axis_index("c")
        base_t = wid * T_PER_W
        for kk, out_hbm in ((0, y0_hbm), (1, y1_hbm)):
            pltpu.sync_copy(
                dest_hbm.at[pl.ds(kk * S + base_t, T_PER_W)], didx_v)
            pltpu.async_copy(y_hbm.at[didx_v], rows_v, sem).wait()
            pltpu.sync_copy(rows_v, out_hbm.at[pl.ds(base_t, T_PER_W)])

    return k(y, dest_flat)


def _final_body(x2_ref, y0_ref, y1_ref, wk_ref, w_ref, b_ref, out_ref):
    w0 = wk_ref[...][:, 0:1]
    w1 = wk_ref[...][:, 1:2]
    z = x2_ref[...] + w0 * y0_ref[...] + w1 * y1_ref[...]
    m = jnp.sum(z, axis=0, keepdims=True) * (1.0 / S)
    out_ref[...] = jnp.dot(m, w_ref[...], precision=HIGHEST,
                           preferred_element_type=F32) + b_ref[...]


def _final(x2, y0, y1, wk, fc_W, fc_b):
    return pl.pallas_call(
        _final_body,
        grid=(1,),
        in_specs=[
            pl.BlockSpec((S, D), lambda i: (0, 0)),
            pl.BlockSpec((S, D), lambda i: (0, 0)),
            pl.BlockSpec((S, D), lambda i: (0, 0)),
            pl.BlockSpec((S, K), lambda i: (0, 0)),
            pl.BlockSpec((D, D), lambda i: (0, 0)),
            pl.BlockSpec((D,), lambda i: (0,)),
        ],
        out_specs=pl.BlockSpec((1, D), lambda i: (0, 0)),
        out_shape=jax.ShapeDtypeStruct((1, D), F32),
    )(x2, y0, y1, wk, fc_W, fc_b)


def kernel(x, qkv_W, qkv_b, attn_out_W, attn_out_b, gate_W, gate_b,
           exp_W1, exp_b1, exp_W2, exp_b2, ln1_w, ln1_b, ln2_w, ln2_b,
           fc_W, fc_b):
    xs = x.reshape(S, D)
    qkv_hi, qkv_lo = _ln_qkv(xs, qkv_W, qkv_b, ln1_w, ln1_b)
    aoT = _attention(qkv_hi, qkv_lo)
    x2, h2f, wk, dest, bexp, blive = _proj_router(
        aoT, xs, attn_out_W, attn_out_b, ln2_w, ln2_b, gate_W, gate_b)
    dest_flat = dest.reshape(K * S)
    xs_sorted = _dispatch(h2f, dest_flat)
    y = _moe_ffn(xs_sorted, bexp.reshape(NBT), blive.reshape(NBT),
                 exp_W1, exp_b1.reshape(E, 1, HID),
                 exp_W2, exp_b2.reshape(E, 1, D))
    y0, y1 = _combine_gather(y, dest_flat)
    return _final(x2, y0, y1, wk, fc_W, fc_b)
